# Initial kernel scaffold; baseline (speedup 1.0000x reference)
import jax
import jax.numpy as jnp
from jax.experimental import pallas as pl


def kernel(x, edge_index, edge_attr, batch, node_emb, edge_lin_w, edge_lin_b, conv_weight, gru_Wih, gru_Whh, gru_bih, gru_bhh, fc1_w, fc1_b, fc2_w, fc2_b, fc3_w, fc3_b):
    raise NotImplementedError("write your pallas kernel here")


# speedup vs baseline: 2.9738x; 2.9738x over previous
import functools

import jax
import jax.numpy as jnp
from jax import lax
from jax.experimental import pallas as pl
from jax.experimental.pallas import tpu as pltpu
from jax.experimental.pallas import tpu_sc as plsc

N = 50000
E = 800000
H = 64
NG = 64
NUM_LAYERS = 3

NCORES = 2
NSUB = 16
HALF = N // NCORES
CHUNK = 128
CHUNKS_PER_SUB = 400
EPAD = NSUB * CHUNKS_PER_SUB * CHUNK
ROWS_PER_SUB = 1568
ACC_ROWS = NSUB * ROWS_PER_SUB

BN = 1000
GRID = N // BN
CODES = 104


NSLOT = 2
GROUP = NSLOT * CHUNK
GROUPS_PER_SUB = CHUNKS_PER_SUB // NSLOT
ROWBYTES = CHUNK * H * 4


def _sc_scatter_body(m_hbm, src_hbm, dst_hbm, out_hbm, *scr):
    srcg_v, dstg_v = scr[0], scr[1]
    idxs = scr[2:2 + NSLOT]
    rows = scr[2 + NSLOT:2 + 2 * NSLOT]
    zrow_v = scr[2 + 2 * NSLOT]
    acc = scr[3 + 2 * NSLOT]
    gsems = scr[4 + 2 * NSLOT:4 + 3 * NSLOT]
    ssems = scr[4 + 3 * NSLOT:4 + 4 * NSLOT]
    c = lax.axis_index("c")
    s = lax.axis_index("s")
    node_base = c * HALF
    row0 = s * ROWS_PER_SUB

    zero16 = jnp.zeros((16,), jnp.float32)

    def zrow_body(i, carry):
        for k in range(H // 16):
            zrow_v[i, pl.ds(k * 16, 16)] = zero16
        return carry

    lax.fori_loop(0, CHUNK, zrow_body, 0)

    for j in range(ROWS_PER_SUB // CHUNK):
        pltpu.sync_copy(zrow_v, acc.at[pl.ds(row0 + j * CHUNK, CHUNK)])
    rem = ROWS_PER_SUB % CHUNK
    if rem:
        pltpu.sync_copy(zrow_v.at[pl.ds(0, rem)],
                        acc.at[pl.ds(row0 + (ROWS_PER_SUB // CHUNK) * CHUNK, rem)])

    plsc.subcore_barrier()

    def outer_body(o, carry):
        gbase = (s * GROUPS_PER_SUB + o) * GROUP
        pltpu.sync_copy(src_hbm.at[pl.ds(gbase, GROUP)], srcg_v)
        pltpu.sync_copy(dst_hbm.at[pl.ds(gbase, GROUP)], dstg_v)
        for b in range(NSLOT):
            @pl.when(o > 0)
            def _(b=b):
                pltpu.make_async_copy(rows[b], acc.at[idxs[b]], ssems[b]).wait()
            pltpu.async_copy(m_hbm.at[srcg_v.at[pl.ds(b * CHUNK, CHUNK)]],
                             rows[b], gsems[b])
        for b in range(NSLOT):
            for k in range(CHUNK // 16):
                d = dstg_v[pl.ds(b * CHUNK + k * 16, 16)]
                loc = d - node_base
                oob = (loc < 0) | (loc >= HALF)
                idxs[b][pl.ds(k * 16, 16)] = jnp.where(oob, HALF, loc)
            pltpu.make_async_copy(m_hbm.at[srcg_v.at[pl.ds(b * CHUNK, CHUNK)]],
                                  rows[b], gsems[b]).wait()
            pltpu.async_copy(rows[b], acc.at[idxs[b]], ssems[b], add=True)
        return carry

    lax.fori_loop(0, GROUPS_PER_SUB, outer_body, 0)

    for b in range(NSLOT):
        pltpu.make_async_copy(rows[b], acc.at[idxs[b]], ssems[b]).wait()

    plsc.subcore_barrier()
    pltpu.sync_copy(acc.at[pl.ds(row0, ROWS_PER_SUB)],
                    out_hbm.at[c, pl.ds(row0, ROWS_PER_SUB)])


def _sc_scatter(m, srcp, dstp):
    mesh = plsc.VectorSubcoreMesh(core_axis_name="c", subcore_axis_name="s")
    f = pl.kernel(
        _sc_scatter_body,
        out_type=jax.ShapeDtypeStruct((NCORES, ACC_ROWS, H), jnp.float32),
        mesh=mesh,
        scratch_types=(
            [pltpu.VMEM((GROUP,), jnp.int32)] * 2
            + [pltpu.VMEM((CHUNK,), jnp.int32)] * NSLOT
            + [pltpu.VMEM((CHUNK, H), jnp.float32)] * NSLOT
            + [pltpu.VMEM((CHUNK, H), jnp.float32)]
            + [pltpu.VMEM_SHARED((ACC_ROWS + 16, H), jnp.float32)]
            + [pltpu.SemaphoreType.DMA] * (2 * NSLOT)
        ),
        compiler_params=pltpu.CompilerParams(use_tc_tiling_on_sc=False),
    )
    return f(m, srcp, dstp)


def _pre_body(x_ref, emb_ref, w1_ref, h0_ref, m1_ref):
    codes = x_ref[...]
    onehot = (codes == lax.broadcasted_iota(jnp.int32, (1, CODES), 1)
              ).astype(jnp.float32)
    h0 = lax.dot_general(onehot, emb_ref[...], (((1,), (0,)), ((), ())),
                         preferred_element_type=jnp.float32)
    h0_ref[...] = h0
    m1_ref[...] = jnp.dot(h0, w1_ref[...], preferred_element_type=jnp.float32)


def _pre(x, emb_pad, w1):
    return pl.pallas_call(
        _pre_body,
        grid=(GRID,),
        in_specs=[
            pl.BlockSpec((BN, 1), lambda i: (i, 0)),
            pl.BlockSpec((CODES, H), lambda i: (0, 0)),
            pl.BlockSpec((H, H), lambda i: (0, 0)),
        ],
        out_specs=[
            pl.BlockSpec((BN, H), lambda i: (i, 0)),
            pl.BlockSpec((BN, H), lambda i: (i, 0)),
        ],
        out_shape=[jax.ShapeDtypeStruct((N, H), jnp.float32)] * 2,
    )(x, emb_pad, w1)


def _gru_body(h_ref, a_ref, wihT, whhT, brz, bn_i, bn_h, wnext,
              hn_ref, mn_ref):
    h = h_ref[...]
    a = a_ref[0]
    gi = jnp.dot(a, wihT[...], preferred_element_type=jnp.float32)
    gh = jnp.dot(h, whhT[...], preferred_element_type=jnp.float32)
    rz = jax.nn.sigmoid(gi[:, :2 * H] + gh[:, :2 * H] + brz[...])
    r = rz[:, :H]
    z = rz[:, H:]
    n = jnp.tanh(gi[:, 2 * H:] + bn_i[...]
                 + r * (gh[:, 2 * H:] + bn_h[...]))
    hn = jax.nn.relu((1.0 - z) * n + z * h)
    hn_ref[...] = hn
    if mn_ref is not None:
        mn_ref[...] = jnp.dot(hn, wnext[...], preferred_element_type=jnp.float32)


def _gru(h, agg, wihT, whhT, brz, bn_i, bn_h, wnext):
    has_next = wnext is not None
    if not has_next:
        wnext = jnp.zeros((H, H), jnp.float32)
    body = (_gru_body if has_next
            else (lambda *refs: _gru_body(*refs, None)))
    per_core = HALF // BN
    out_shape = [jax.ShapeDtypeStruct((N, H), jnp.float32)]
    out_specs = [pl.BlockSpec((BN, H), lambda i: (i, 0))]
    if has_next:
        out_shape.append(jax.ShapeDtypeStruct((N, H), jnp.float32))
        out_specs.append(pl.BlockSpec((BN, H), lambda i: (i, 0)))
    return pl.pallas_call(
        body,
        grid=(GRID,),
        in_specs=[
            pl.BlockSpec((BN, H), lambda i: (i, 0)),
            pl.BlockSpec((1, BN, H), lambda i: (i // per_core, i % per_core, 0)),
            pl.BlockSpec((H, 3 * H), lambda i: (0, 0)),
            pl.BlockSpec((H, 3 * H), lambda i: (0, 0)),
            pl.BlockSpec((1, 2 * H), lambda i: (0, 0)),
            pl.BlockSpec((1, H), lambda i: (0, 0)),
            pl.BlockSpec((1, H), lambda i: (0, 0)),
            pl.BlockSpec((H, H), lambda i: (0, 0)),
        ],
        out_specs=out_specs,
        out_shape=out_shape,
    )(h, agg, wihT, whhT, brz, bn_i, bn_h, wnext)


BNP = 2000
GRIDP = N // BNP


def _pool_body(batch_ref, h_ref, f1w, f1b, f2w, f2b, f3w, f3b,
               out_ref, sums, cnt):
    i = pl.program_id(0)

    @pl.when(i == 0)
    def _():
        sums[...] = jnp.zeros_like(sums)
        cnt[...] = jnp.zeros_like(cnt)

    onehot = (batch_ref[...] == lax.broadcasted_iota(jnp.int32, (1, NG), 1)
              ).astype(jnp.float32)
    h = h_ref[...]
    sums[...] += lax.dot_general(onehot, h, (((0,), (0,)), ((), ())),
                                 preferred_element_type=jnp.float32)
    cnt[...] += lax.dot_general(onehot, jnp.ones((BNP, 1), jnp.float32),
                                (((0,), (0,)), ((), ())),
                                preferred_element_type=jnp.float32)

    @pl.when(i == GRIDP - 1)
    def _():
        pooled = sums[...] / jnp.maximum(cnt[...], 1.0)
        o = jax.nn.relu(jnp.dot(pooled, f1w[...],
                                preferred_element_type=jnp.float32) + f1b[...])
        o = jax.nn.relu(jnp.dot(o, f2w[...],
                                preferred_element_type=jnp.float32) + f2b[...])
        o = jnp.dot(o, f3w[...], preferred_element_type=jnp.float32) + f3b[...]
        out_ref[...] = o


def _pool(batch2d, h, f1w, f1b, f2w, f2b, f3w, f3b):
    return pl.pallas_call(
        _pool_body,
        grid=(GRIDP,),
        in_specs=[
            pl.BlockSpec((BNP, 1), lambda i: (i, 0)),
            pl.BlockSpec((BNP, H), lambda i: (i, 0)),
            pl.BlockSpec((H, H // 2), lambda i: (0, 0)),
            pl.BlockSpec((1, H // 2), lambda i: (0, 0)),
            pl.BlockSpec((H // 2, H // 4), lambda i: (0, 0)),
            pl.BlockSpec((1, H // 4), lambda i: (0, 0)),
            pl.BlockSpec((H // 4, 1), lambda i: (0, 0)),
            pl.BlockSpec((1, 1), lambda i: (0, 0)),
        ],
        out_specs=pl.BlockSpec((NG, 1), lambda i: (0, 0)),
        out_shape=jax.ShapeDtypeStruct((NG, 1), jnp.float32),
        scratch_shapes=[
            pltpu.VMEM((NG, NG), jnp.float32),
            pltpu.VMEM((NG, 1), jnp.float32),
        ],
    )(batch2d, h, f1w, f1b, f2w, f2b, f3w, f3b)


def kernel(x, edge_index, edge_attr, batch, node_emb, edge_lin_w, edge_lin_b,
           conv_weight, gru_Wih, gru_Whh, gru_bih, gru_bhh,
           fc1_w, fc1_b, fc2_w, fc2_b, fc3_w, fc3_b):
    src = edge_index[0]
    dst = edge_index[1]
    pad = EPAD - E
    srcp = jnp.concatenate([src, jnp.zeros((pad,), jnp.int32)])
    dstp = jnp.concatenate([dst, jnp.full((pad,), -1, jnp.int32)])

    emb_pad = jnp.pad(node_emb, ((0, CODES - node_emb.shape[0]), (0, 0)))

    h, m = _pre(x, emb_pad, conv_weight[0])

    for i in range(NUM_LAYERS):
        agg = _sc_scatter(m, srcp, dstp)
        wihT = gru_Wih[i].T
        whhT = gru_Whh[i].T
        brz = (gru_bih[i, :2 * H] + gru_bhh[i, :2 * H]).reshape(1, 2 * H)
        bn_i = gru_bih[i, 2 * H:].reshape(1, H)
        bn_h = gru_bhh[i, 2 * H:].reshape(1, H)
        wnext = conv_weight[i + 1] if i + 1 < NUM_LAYERS else None
        res = _gru(h, agg, wihT, whhT, brz, bn_i, bn_h, wnext)
        if wnext is not None:
            h, m = res
        else:
            h = res[0]

    out = _pool(batch.reshape(N, 1), h,
                fc1_w.T, fc1_b.reshape(1, H // 2),
                fc2_w.T, fc2_b.reshape(1, H // 4),
                fc3_w.T, fc3_b.reshape(1, 1))
    return out[:, 0]
